# Initial kernel scaffold; baseline (speedup 1.0000x reference)
import jax
import jax.numpy as jnp
from jax.experimental import pallas as pl


def kernel(hidden_states, chord_changes, W1, b1, W2, b2, gamma, beta):
    raise NotImplementedError("write your pallas kernel here")


# speedup vs baseline: 2.5904x; 2.5904x over previous
import jax
import jax.numpy as jnp
from jax import lax
from jax.experimental import pallas as pl
from jax.experimental.pallas import tpu as pltpu


def _seg_scan_fwd(v, f):
    T = v.shape[0]
    k = 1
    while k < T:
        vz = jnp.zeros((k, v.shape[1]), v.dtype)
        fz = jnp.zeros((k, 1), f.dtype)
        vs = jnp.concatenate([vz, v[:-k]], axis=0)
        fs = jnp.concatenate([fz, f[:-k]], axis=0)
        v = v + f * vs
        f = f * fs
        k *= 2
    return v


def _seg_scan_bwd(v, g):
    T = v.shape[0]
    k = 1
    while k < T:
        vz = jnp.zeros((k, v.shape[1]), v.dtype)
        gz = jnp.zeros((k, 1), g.dtype)
        vs = jnp.concatenate([v[k:], vz], axis=0)
        gs = jnp.concatenate([g[k:], gz], axis=0)
        v = v + g * vs
        g = g * gs
        k *= 2
    return v


def _body(cc_ref, x_ref, w1_ref, b1_ref, w2_ref, b2_ref, gm_ref, bt_ref,
          o_ref):
    x = x_ref[0]
    T, D = x.shape
    F = w1_ref.shape[1]

    cc = cc_ref[0]
    t_iota = lax.broadcasted_iota(jnp.int32, (T, 1), 0)
    m = jnp.where((cc == 0) & (t_iota > 0), 1.0, 0.0).astype(jnp.float32)
    g = jnp.concatenate([m[1:], jnp.zeros((1, 1), jnp.float32)], axis=0)

    ones = jnp.ones((T, 1), jnp.float32)
    vF = _seg_scan_fwd(x, m)
    cF = _seg_scan_fwd(ones, m)
    vB = _seg_scan_bwd(x, g)
    cB = _seg_scan_bwd(ones, g)

    tot = vF + vB - x
    cnt = cF + cB - 1.0
    xm = tot / cnt

    acc = xm
    CH = 512
    for j in range(0, F, CH):
        w1c = w1_ref[:, j:j + CH]
        b1c = b1_ref[:, j:j + CH]
        h1 = jnp.maximum(
            jnp.dot(xm, w1c, preferred_element_type=jnp.float32) + b1c, 0.0)
        w2c = w2_ref[j:j + CH, :]
        acc = acc + jnp.dot(h1, w2c, preferred_element_type=jnp.float32)
    acc = acc + b2_ref[...]

    mu = jnp.mean(acc, axis=-1, keepdims=True)
    d = acc - mu
    var = jnp.mean(d * d, axis=-1, keepdims=True)
    out = gm_ref[...] * d * lax.rsqrt(var + 1e-3) + bt_ref[...]
    o_ref[0] = out


def kernel(hidden_states, chord_changes, W1, b1, W2, b2, gamma, beta):
    B, T, D = hidden_states.shape
    F = W1.shape[1]
    cc3 = chord_changes.reshape(B, T, 1)

    grid = (B,)
    out = pl.pallas_call(
        _body,
        grid=grid,
        in_specs=[
            pl.BlockSpec((1, T, 1), lambda b: (b, 0, 0)),
            pl.BlockSpec((1, T, D), lambda b: (b, 0, 0)),
            pl.BlockSpec((D, F), lambda b: (0, 0)),
            pl.BlockSpec((1, F), lambda b: (0, 0)),
            pl.BlockSpec((F, D), lambda b: (0, 0)),
            pl.BlockSpec((1, D), lambda b: (0, 0)),
            pl.BlockSpec((1, D), lambda b: (0, 0)),
            pl.BlockSpec((1, D), lambda b: (0, 0)),
        ],
        out_specs=pl.BlockSpec((1, T, D), lambda b: (b, 0, 0)),
        out_shape=jax.ShapeDtypeStruct((B, T, D), jnp.float32),
    )(cc3, hidden_states, W1, b1.reshape(1, F), W2, b2.reshape(1, D),
      gamma.reshape(1, D), beta.reshape(1, D))
    return out
